# baseline (speedup 1.0000x reference)
import jax
import jax.numpy as jnp
from jax import lax
from jax.experimental import pallas as pl
from jax.experimental.pallas import tpu as pltpu
from jax.experimental.pallas import tpu_sc as plsc

N = 10000
NP = 10240
E = 320000
D = 128
EPS = 0.01

NC = 2
NS = 16
NW = NC * NS
C = 40
G = 10
NCH = E // (NW * C)
NGR = NCH // G

EPW = E // NW
RPT = NP // NS

_f32 = jnp.float32
_mesh = plsc.VectorSubcoreMesh(core_axis_name="c", subcore_axis_name="s")


def _tables_body_emb(x_ref, ew_ref, eb_ref, lw_ref, wa_ref, wb_ref, b1_ref,
                     xe_ref, s_ref, t_ref):
    xe = jnp.dot(x_ref[...], ew_ref[...].T,
                 preferred_element_type=_f32) + eb_ref[...]
    xe_ref[...] = xe
    s_ref[:, :D] = jnp.dot(xe, lw_ref[...].T, preferred_element_type=_f32)
    s_ref[:, D:] = jnp.dot(xe, wa_ref[...].T,
                           preferred_element_type=_f32) + b1_ref[...]
    t_ref[...] = jnp.dot(xe, wb_ref[...].T, preferred_element_type=_f32)


def _tables_body(x_ref, lw_ref, wa_ref, wb_ref, b1_ref, s_ref, t_ref):
    xe = x_ref[...]
    s_ref[:, :D] = jnp.dot(xe, lw_ref[...].T, preferred_element_type=_f32)
    s_ref[:, D:] = jnp.dot(xe, wa_ref[...].T,
                           preferred_element_type=_f32) + b1_ref[...]
    t_ref[...] = jnp.dot(xe, wb_ref[...].T, preferred_element_type=_f32)


_BN = 1000


def _w_spec():
    return pl.BlockSpec((D, D), lambda i: (0, 0))


def _b_spec():
    return pl.BlockSpec((1, D), lambda i: (0, 0))


def _tc_tables_emb(x, emb_w, emb_b, lin_w, wa, wb, b1):
    return pl.pallas_call(
        _tables_body_emb,
        grid=(N // _BN,),
        in_specs=[pl.BlockSpec((_BN, D), lambda i: (i, 0)),
                  _w_spec(), _b_spec(), _w_spec(), _w_spec(), _w_spec(),
                  _b_spec()],
        out_specs=[pl.BlockSpec((_BN, D), lambda i: (i, 0)),
                   pl.BlockSpec((_BN, 2 * D), lambda i: (i, 0)),
                   pl.BlockSpec((_BN, D), lambda i: (i, 0))],
        out_shape=[jax.ShapeDtypeStruct((N, D), _f32),
                   jax.ShapeDtypeStruct((N, 2 * D), _f32),
                   jax.ShapeDtypeStruct((N, D), _f32)],
    )(x, emb_w, emb_b, lin_w, wa, wb, b1)


def _tc_tables(x, lin_w, wa, wb, b1):
    return pl.pallas_call(
        _tables_body,
        grid=(N // _BN,),
        in_specs=[pl.BlockSpec((_BN, D), lambda i: (i, 0)),
                  _w_spec(), _w_spec(), _w_spec(), _b_spec()],
        out_specs=[pl.BlockSpec((_BN, 2 * D), lambda i: (i, 0)),
                   pl.BlockSpec((_BN, D), lambda i: (i, 0))],
        out_shape=[jax.ShapeDtypeStruct((N, 2 * D), _f32),
                   jax.ShapeDtypeStruct((N, D), _f32)],
    )(x, lin_w, wa, wb, b1)


def _update_body(x_ref, pq_ref, w1_ref, bb1_ref, w2_ref, bb2_ref, out_ref):
    pq = pq_ref[...]
    conv = pq[0] + pq[1]
    x1 = x_ref[...] - (EPS * EPS) * conv
    t = jnp.tanh(jnp.dot(x1, w1_ref[...].T,
                         preferred_element_type=_f32) + bb1_ref[...])
    out_ref[...] = jnp.dot(t, w2_ref[...].T,
                           preferred_element_type=_f32) + bb2_ref[...]


def _update_body_ro(x_ref, pq_ref, w1_ref, bb1_ref, w2_ref, bb2_ref,
                    rw_ref, rb_ref, out_ref):
    pq = pq_ref[...]
    conv = pq[0] + pq[1]
    x1 = x_ref[...] - (EPS * EPS) * conv
    t = jnp.tanh(jnp.dot(x1, w1_ref[...].T,
                         preferred_element_type=_f32) + bb1_ref[...])
    h = jnp.dot(t, w2_ref[...].T, preferred_element_type=_f32) + bb2_ref[...]
    out_ref[...] = jnp.dot(h, rw_ref[...].T,
                           preferred_element_type=_f32) + rb_ref[...]


def _pq_spec():
    return pl.BlockSpec((2, _BN, D), lambda i: (0, i, 0))


def _tc_update(x, pq, w1, b1, w2, b2):
    return pl.pallas_call(
        _update_body,
        grid=(N // _BN,),
        in_specs=[pl.BlockSpec((_BN, D), lambda i: (i, 0)), _pq_spec(),
                  _w_spec(), _b_spec(), _w_spec(), _b_spec()],
        out_specs=pl.BlockSpec((_BN, D), lambda i: (i, 0)),
        out_shape=jax.ShapeDtypeStruct((N, D), _f32),
    )(x, pq, w1, b1, w2, b2)


def _tc_update_ro(x, pq, w1, b1, w2, b2, rw, rb):
    return pl.pallas_call(
        _update_body_ro,
        grid=(N // _BN,),
        in_specs=[pl.BlockSpec((_BN, D), lambda i: (i, 0)), _pq_spec(),
                  _w_spec(), _b_spec(), _w_spec(), _b_spec(),
                  _w_spec(), _b_spec()],
        out_specs=pl.BlockSpec((_BN, D), lambda i: (i, 0)),
        out_shape=jax.ShapeDtypeStruct((N, D), _f32),
    )(x, pq, w1, b1, w2, b2, rw, rb)


def _fused_body(s_hbm, t_hbm, idx_hbm, w2_hbm, b2_hbm, out_hbm,
                cidx, sbuf0, sbuf1, tbuf0, tbuf1, mnbuf, erbuf,
                w2buf, b2buf, sem0, sem1, sem2, sem3, sem4, acc):
    c = lax.axis_index("c")
    sid = lax.axis_index("s")
    wid = sid * NC + c

    pltpu.sync_copy(w2_hbm, w2buf)
    pltpu.sync_copy(b2_hbm, b2buf)

    def zrow(r, carry):
        for k in range(8):
            mnbuf[r, pl.ds(k * 16, 16)] = jnp.zeros((16,), _f32)
        return carry

    lax.fori_loop(0, 2 * C, zrow, 0)
    r0 = sid * RPT

    def zcopy(i, carry):
        pltpu.sync_copy(mnbuf, acc.at[pl.ds(r0 + i * 2 * C, 2 * C)])
        return carry

    lax.fori_loop(0, RPT // (2 * C), zcopy, 0)
    plsc.subcore_barrier()

    lanes = lax.iota(jnp.int32, 16)

    def er_pass(sb, tb):
        @plsc.parallel_loop(0, C, 1, unroll=8)
        def edge(r):
            acc16 = jnp.zeros((16,), _f32)
            for k in range(8):
                av = sb[r, pl.ds(D + k * 16, 16)]
                bv = tb[r, pl.ds(k * 16, 16)]
                tv = jnp.maximum(av + bv, 0.0)
                acc16 = acc16 + tv * w2buf[pl.ds(k * 16, 16)]
            tot = acc16
            for s in (8, 4, 2, 1):
                tot = tot + tot.at[lanes ^ s].get(mode="promise_in_bounds")
            erbuf[r, :] = jnp.abs(tot + b2buf[...])

    def m_pass(sb):
        @plsc.parallel_loop(0, C, 1, unroll=8)
        def edge(r):
            er16 = erbuf[r, :]
            for k in range(8):
                fv = sb[r, pl.ds(k * 16, 16)]
                mv = er16 * fv
                mnbuf[r, pl.ds(k * 16, 16)] = mv
                mnbuf[C + r, pl.ds(k * 16, 16)] = -mv

    bufs = ((sbuf0, tbuf0, sem0, sem1), (sbuf1, tbuf1, sem2, sem3))

    def issue(k, par):
        sb, tb, sa, sb_sem = bufs[par]
        a = pltpu.async_copy(s_hbm.at[cidx.at[k, pl.ds(0, C)]], sb, sa)
        b = pltpu.async_copy(t_hbm.at[cidx.at[k, pl.ds(C, C)]], tb, sb_sem)
        return a, b

    def group(g, carry):
        grow = wid * NGR + g
        pltpu.sync_copy(idx_hbm.at[grow], cidx)
        descs = issue(0, 0)
        sdesc = None
        for k in range(G):
            par = k % 2
            cur = descs
            if k < G - 1:
                descs = issue(k + 1, 1 - par)
            cur[0].wait()
            cur[1].wait()
            er_pass(bufs[par][0], bufs[par][1])
            if sdesc is not None:
                sdesc.wait()
            m_pass(bufs[par][0])
            sdesc = pltpu.async_copy(mnbuf, acc.at[cidx.at[k]], sem4,
                                     add=True)
        sdesc.wait()
        return carry

    lax.fori_loop(0, NGR, group, 0)
    plsc.subcore_barrier()

    pltpu.sync_copy(acc.at[pl.ds(r0, RPT)],
                    out_hbm.at[pl.ds(c * NP + r0, RPT)])


_sc_fused = pl.kernel(
    _fused_body,
    out_type=jax.ShapeDtypeStruct((2 * NP, D), _f32),
    mesh=_mesh,
    scratch_types=[pltpu.VMEM((G, 2 * C), jnp.int32),
                   pltpu.VMEM((C, 2 * D), _f32),
                   pltpu.VMEM((C, 2 * D), _f32),
                   pltpu.VMEM((C, D), _f32),
                   pltpu.VMEM((C, D), _f32),
                   pltpu.VMEM((2 * C, D), _f32),
                   pltpu.VMEM((C, 16), _f32),
                   pltpu.VMEM((D,), _f32),
                   pltpu.VMEM((16,), _f32),
                   pltpu.SemaphoreType.DMA,
                   pltpu.SemaphoreType.DMA,
                   pltpu.SemaphoreType.DMA,
                   pltpu.SemaphoreType.DMA,
                   pltpu.SemaphoreType.DMA,
                   pltpu.VMEM_SHARED((NP, D), _f32)],
)


def _block(x, src, dst, lin_w, er1_w, er1_b, er2_w, er2_b,
           mlp1_w, mlp1_b, mlp2_w, mlp2_b, emb=None, readout=None):
    wa = er1_w[:, :D]
    wb = er1_w[:, D:]
    b1 = er1_b.reshape(1, D)
    if emb is not None:
        emb_w, emb_b = emb
        xe, s, t = _tc_tables_emb(x, emb_w, emb_b.reshape(1, D),
                                  lin_w, wa, wb, b1)
    else:
        xe = x
        s, t = _tc_tables(x, lin_w, wa, wb, b1)
    w2 = er2_w.reshape(D)
    b2v = jnp.full((16,), er2_b[0], _f32)
    s3 = src.reshape(NW * NGR, G, C)
    d3 = dst.reshape(NW * NGR, G, C)
    icat = jnp.concatenate([s3, d3], axis=2)
    pq = _sc_fused(s, t, icat, w2, b2v).reshape(2, NP, D)[:, :N, :]
    if readout is None:
        return _tc_update(xe, pq, mlp1_w, mlp1_b.reshape(1, D),
                          mlp2_w, mlp2_b.reshape(1, D))
    rw, rb = readout
    return _tc_update_ro(xe, pq, mlp1_w, mlp1_b.reshape(1, D),
                         mlp2_w, mlp2_b.reshape(1, D), rw, rb.reshape(1, D))


def kernel(x, edge_index, emb_w, emb_b,
           lin_w0, diss_w0, diss_b0, er1_w0, er1_b0, er2_w0, er2_b0,
           mlp1_w0, mlp1_b0, mlp2_w0, mlp2_b0,
           lin_w1, diss_w1, diss_b1, er1_w1, er1_b1, er2_w1, er2_b1,
           mlp1_w1, mlp1_b1, mlp2_w1, mlp2_b1,
           readout_w, readout_b):
    src = edge_index[0]
    dst = edge_index[1]
    h = _block(x, src, dst, lin_w0, er1_w0, er1_b0, er2_w0, er2_b0,
               mlp1_w0, mlp1_b0, mlp2_w0, mlp2_b0, emb=(emb_w, emb_b))
    return _block(h, src, dst, lin_w1, er1_w1, er1_b1, er2_w1, er2_b1,
                  mlp1_w1, mlp1_b1, mlp2_w1, mlp2_b1,
                  readout=(readout_w, readout_b))


# speedup vs baseline: 1.1417x; 1.1417x over previous
import jax
import jax.numpy as jnp
from jax import lax
from jax.experimental import pallas as pl
from jax.experimental.pallas import tpu as pltpu
from jax.experimental.pallas import tpu_sc as plsc

N = 10000
NP = 10240
E = 320000
D = 128
EPS = 0.01

NC = 2
NS = 16
NW = NC * NS
C = 40
G = 10
NCH = E // (NW * C)
NGR = NCH // G

EPW = E // NW
RPT = NP // NS

_f32 = jnp.float32
_mesh = plsc.VectorSubcoreMesh(core_axis_name="c", subcore_axis_name="s")


def _tables_body_emb(x_ref, ew_ref, eb_ref, lw_ref, wa_ref, wb_ref, b1_ref,
                     xe_ref, s_ref, t_ref):
    xe = jnp.dot(x_ref[...], ew_ref[...].T,
                 preferred_element_type=_f32) + eb_ref[...]
    xe_ref[...] = xe
    s_ref[:, :D] = jnp.dot(xe, lw_ref[...].T, preferred_element_type=_f32)
    s_ref[:, D:] = jnp.dot(xe, wa_ref[...].T,
                           preferred_element_type=_f32) + b1_ref[...]
    t_ref[...] = jnp.dot(xe, wb_ref[...].T, preferred_element_type=_f32)


def _tables_body(x_ref, lw_ref, wa_ref, wb_ref, b1_ref, s_ref, t_ref):
    xe = x_ref[...]
    s_ref[:, :D] = jnp.dot(xe, lw_ref[...].T, preferred_element_type=_f32)
    s_ref[:, D:] = jnp.dot(xe, wa_ref[...].T,
                           preferred_element_type=_f32) + b1_ref[...]
    t_ref[...] = jnp.dot(xe, wb_ref[...].T, preferred_element_type=_f32)


_BN = 1000


def _w_spec():
    return pl.BlockSpec((D, D), lambda i: (0, 0))


def _b_spec():
    return pl.BlockSpec((1, D), lambda i: (0, 0))


def _tc_tables_emb(x, emb_w, emb_b, lin_w, wa, wb, b1):
    return pl.pallas_call(
        _tables_body_emb,
        grid=(N // _BN,),
        in_specs=[pl.BlockSpec((_BN, D), lambda i: (i, 0)),
                  _w_spec(), _b_spec(), _w_spec(), _w_spec(), _w_spec(),
                  _b_spec()],
        out_specs=[pl.BlockSpec((_BN, D), lambda i: (i, 0)),
                   pl.BlockSpec((_BN, 2 * D), lambda i: (i, 0)),
                   pl.BlockSpec((_BN, D), lambda i: (i, 0))],
        out_shape=[jax.ShapeDtypeStruct((N, D), _f32),
                   jax.ShapeDtypeStruct((N, 2 * D), _f32),
                   jax.ShapeDtypeStruct((N, D), _f32)],
    )(x, emb_w, emb_b, lin_w, wa, wb, b1)


def _tc_tables(x, lin_w, wa, wb, b1):
    return pl.pallas_call(
        _tables_body,
        grid=(N // _BN,),
        in_specs=[pl.BlockSpec((_BN, D), lambda i: (i, 0)),
                  _w_spec(), _w_spec(), _w_spec(), _b_spec()],
        out_specs=[pl.BlockSpec((_BN, 2 * D), lambda i: (i, 0)),
                   pl.BlockSpec((_BN, D), lambda i: (i, 0))],
        out_shape=[jax.ShapeDtypeStruct((N, 2 * D), _f32),
                   jax.ShapeDtypeStruct((N, D), _f32)],
    )(x, lin_w, wa, wb, b1)


def _update_body(x_ref, pq_ref, w1_ref, bb1_ref, w2_ref, bb2_ref, out_ref):
    pq = pq_ref[...]
    conv = pq[0] + pq[1]
    x1 = x_ref[...] - (EPS * EPS) * conv
    t = jnp.tanh(jnp.dot(x1, w1_ref[...].T,
                         preferred_element_type=_f32) + bb1_ref[...])
    out_ref[...] = jnp.dot(t, w2_ref[...].T,
                           preferred_element_type=_f32) + bb2_ref[...]


def _update_body_ro(x_ref, pq_ref, w1_ref, bb1_ref, w2_ref, bb2_ref,
                    rw_ref, rb_ref, out_ref):
    pq = pq_ref[...]
    conv = pq[0] + pq[1]
    x1 = x_ref[...] - (EPS * EPS) * conv
    t = jnp.tanh(jnp.dot(x1, w1_ref[...].T,
                         preferred_element_type=_f32) + bb1_ref[...])
    h = jnp.dot(t, w2_ref[...].T, preferred_element_type=_f32) + bb2_ref[...]
    out_ref[...] = jnp.dot(h, rw_ref[...].T,
                           preferred_element_type=_f32) + rb_ref[...]


def _pq_spec():
    return pl.BlockSpec((2, _BN, D), lambda i: (0, i, 0))


def _tc_update(x, pq, w1, b1, w2, b2):
    return pl.pallas_call(
        _update_body,
        grid=(N // _BN,),
        in_specs=[pl.BlockSpec((_BN, D), lambda i: (i, 0)), _pq_spec(),
                  _w_spec(), _b_spec(), _w_spec(), _b_spec()],
        out_specs=pl.BlockSpec((_BN, D), lambda i: (i, 0)),
        out_shape=jax.ShapeDtypeStruct((N, D), _f32),
    )(x, pq, w1, b1, w2, b2)


def _tc_update_ro(x, pq, w1, b1, w2, b2, rw, rb):
    return pl.pallas_call(
        _update_body_ro,
        grid=(N // _BN,),
        in_specs=[pl.BlockSpec((_BN, D), lambda i: (i, 0)), _pq_spec(),
                  _w_spec(), _b_spec(), _w_spec(), _b_spec(),
                  _w_spec(), _b_spec()],
        out_specs=pl.BlockSpec((_BN, D), lambda i: (i, 0)),
        out_shape=jax.ShapeDtypeStruct((N, D), _f32),
    )(x, pq, w1, b1, w2, b2, rw, rb)


def _fused_body(s_hbm, t_hbm, idx_hbm, w2_hbm, b2_hbm, out_hbm,
                cidx, sbuf0, sbuf1, tbuf0, tbuf1, mnbuf, erbuf,
                w2buf, b2buf, sem0, sem1, sem2, sem3, sem4, acc):
    c = lax.axis_index("c")
    sid = lax.axis_index("s")
    wid = sid * NC + c

    pltpu.sync_copy(w2_hbm, w2buf)
    pltpu.sync_copy(b2_hbm, b2buf)

    def zrow(r, carry):
        for k in range(8):
            mnbuf[r, pl.ds(k * 16, 16)] = jnp.zeros((16,), _f32)
        return carry

    lax.fori_loop(0, 2 * C, zrow, 0)
    r0 = sid * RPT

    def zcopy(i, carry):
        pltpu.sync_copy(mnbuf, acc.at[pl.ds(r0 + i * 2 * C, 2 * C)])
        return carry

    lax.fori_loop(0, RPT // (2 * C), zcopy, 0)
    plsc.subcore_barrier()

    lanes = lax.iota(jnp.int32, 16)

    def er_pass(sb, tb):
        @plsc.parallel_loop(0, C, 1, unroll=4)
        def edge(r):
            acc16 = jnp.zeros((16,), _f32)
            for k in range(8):
                av = sb[r, pl.ds(D + k * 16, 16)]
                bv = tb[r, pl.ds(k * 16, 16)]
                tv = jnp.maximum(av + bv, 0.0)
                acc16 = acc16 + tv * w2buf[pl.ds(k * 16, 16)]
            tot = acc16
            for s in (8, 4, 2, 1):
                tot = tot + tot.at[lanes ^ s].get(mode="promise_in_bounds")
            erbuf[r, :] = jnp.abs(tot + b2buf[...])

    def m_pass(sb):
        @plsc.parallel_loop(0, C, 1, unroll=4)
        def edge(r):
            er16 = erbuf[r, :]
            for k in range(8):
                fv = sb[r, pl.ds(k * 16, 16)]
                mv = er16 * fv
                mnbuf[r, pl.ds(k * 16, 16)] = mv
                mnbuf[C + r, pl.ds(k * 16, 16)] = -mv

    bufs = ((sbuf0, tbuf0, sem0, sem1), (sbuf1, tbuf1, sem2, sem3))

    def issue(k, par):
        sb, tb, sa, sb_sem = bufs[par]
        a = pltpu.async_copy(s_hbm.at[cidx.at[k, pl.ds(0, C)]], sb, sa)
        b = pltpu.async_copy(t_hbm.at[cidx.at[k, pl.ds(C, C)]], tb, sb_sem)
        return a, b

    def group(g, carry):
        grow = wid * NGR + g
        pltpu.sync_copy(idx_hbm.at[grow], cidx)
        descs = issue(0, 0)
        sdesc = None
        for k in range(G):
            par = k % 2
            cur = descs
            if k < G - 1:
                descs = issue(k + 1, 1 - par)
            cur[0].wait()
            cur[1].wait()
            er_pass(bufs[par][0], bufs[par][1])
            if sdesc is not None:
                sdesc.wait()
            m_pass(bufs[par][0])
            sdesc = pltpu.async_copy(mnbuf, acc.at[cidx.at[k]], sem4,
                                     add=True)
        sdesc.wait()
        return carry

    lax.fori_loop(0, NGR, group, 0)
    plsc.subcore_barrier()

    pltpu.sync_copy(acc.at[pl.ds(r0, RPT)],
                    out_hbm.at[pl.ds(c * NP + r0, RPT)])


_sc_fused = pl.kernel(
    _fused_body,
    out_type=jax.ShapeDtypeStruct((2 * NP, D), _f32),
    mesh=_mesh,
    scratch_types=[pltpu.VMEM((G, 2 * C), jnp.int32),
                   pltpu.VMEM((C, 2 * D), _f32),
                   pltpu.VMEM((C, 2 * D), _f32),
                   pltpu.VMEM((C, D), _f32),
                   pltpu.VMEM((C, D), _f32),
                   pltpu.VMEM((2 * C, D), _f32),
                   pltpu.VMEM((C, 16), _f32),
                   pltpu.VMEM((D,), _f32),
                   pltpu.VMEM((16,), _f32),
                   pltpu.SemaphoreType.DMA,
                   pltpu.SemaphoreType.DMA,
                   pltpu.SemaphoreType.DMA,
                   pltpu.SemaphoreType.DMA,
                   pltpu.SemaphoreType.DMA,
                   pltpu.VMEM_SHARED((NP, D), _f32)],
)


def _block(x, src, dst, lin_w, er1_w, er1_b, er2_w, er2_b,
           mlp1_w, mlp1_b, mlp2_w, mlp2_b, emb=None, readout=None):
    wa = er1_w[:, :D]
    wb = er1_w[:, D:]
    b1 = er1_b.reshape(1, D)
    if emb is not None:
        emb_w, emb_b = emb
        xe, s, t = _tc_tables_emb(x, emb_w, emb_b.reshape(1, D),
                                  lin_w, wa, wb, b1)
    else:
        xe = x
        s, t = _tc_tables(x, lin_w, wa, wb, b1)
    w2 = er2_w.reshape(D)
    b2v = jnp.full((16,), er2_b[0], _f32)
    s3 = src.reshape(NW * NGR, G, C)
    d3 = dst.reshape(NW * NGR, G, C)
    icat = jnp.concatenate([s3, d3], axis=2)
    pq = _sc_fused(s, t, icat, w2, b2v).reshape(2, NP, D)[:, :N, :]
    if readout is None:
        return _tc_update(xe, pq, mlp1_w, mlp1_b.reshape(1, D),
                          mlp2_w, mlp2_b.reshape(1, D))
    rw, rb = readout
    return _tc_update_ro(xe, pq, mlp1_w, mlp1_b.reshape(1, D),
                         mlp2_w, mlp2_b.reshape(1, D), rw, rb.reshape(1, D))


def kernel(x, edge_index, emb_w, emb_b,
           lin_w0, diss_w0, diss_b0, er1_w0, er1_b0, er2_w0, er2_b0,
           mlp1_w0, mlp1_b0, mlp2_w0, mlp2_b0,
           lin_w1, diss_w1, diss_b1, er1_w1, er1_b1, er2_w1, er2_b1,
           mlp1_w1, mlp1_b1, mlp2_w1, mlp2_b1,
           readout_w, readout_b):
    src = edge_index[0]
    dst = edge_index[1]
    h = _block(x, src, dst, lin_w0, er1_w0, er1_b0, er2_w0, er2_b0,
               mlp1_w0, mlp1_b0, mlp2_w0, mlp2_b0, emb=(emb_w, emb_b))
    return _block(h, src, dst, lin_w1, er1_w1, er1_b1, er2_w1, er2_b1,
                  mlp1_w1, mlp1_b1, mlp2_w1, mlp2_b1,
                  readout=(readout_w, readout_b))
